# baseline (speedup 1.0000x reference)
import jax
import jax.numpy as jnp
from jax.experimental import pallas as pl


def kernel(values, index):
    idx_t = index.T
    n_rows, n_cols = idx_t.shape
    bc = 8192

    def body(v_ref, i_ref, o_ref):
        t = jnp.pad(v_ref[...], (0, 128 - values.shape[0]))
        tb = jnp.broadcast_to(t.reshape(1, 128), (n_rows, 128))
        o_ref[...] = jnp.take_along_axis(tb, i_ref[...], axis=1)

    out_t = pl.pallas_call(
        body,
        grid=(n_cols // bc,),
        in_specs=[
            pl.BlockSpec((values.shape[0],), lambda j: (0,)),
            pl.BlockSpec((n_rows, bc), lambda j: (0, j)),
        ],
        out_specs=pl.BlockSpec((n_rows, bc), lambda j: (0, j)),
        out_shape=jax.ShapeDtypeStruct((n_rows, n_cols), jnp.float32),
    )(values, idx_t)
    return out_t.T
